# baseline (speedup 1.0000x reference)
import functools

import jax
import jax.numpy as jnp
from jax import lax
from jax.experimental import pallas as pl
from jax.experimental.pallas import tpu as pltpu
from jax.experimental.pallas import tpu_sc as plsc

V = 1000000
D = 64
H = 128
B = 1024
T = 50

NC = 2
NS = 16
NW = NC * NS
N = B * T
BPW = N // NW
CH = 80
NCH = BPW // CH


def _sc_gather(table, idx3):
    mesh = plsc.VectorSubcoreMesh(core_axis_name="c", subcore_axis_name="s")

    @functools.partial(
        pl.kernel,
        mesh=mesh,
        out_type=jax.ShapeDtypeStruct((N, D), jnp.float32),
        scratch_types=[
            pltpu.VMEM((NCH, CH), jnp.int32),
            pltpu.VMEM((CH, D), jnp.float32),
            pltpu.SemaphoreType.DMA,
        ],
    )
    def gather_kernel(idx_hbm, table_hbm, out_hbm, idx_v, rows_v, sem):
        wid = lax.axis_index("s") * NC + lax.axis_index("c")
        base = wid * BPW
        pltpu.sync_copy(idx_hbm.at[wid], idx_v)

        def chunk(j):
            for k in range(CH // 16):
                vec = idx_v[j, pl.ds(k * 16, 16)]
                for l in range(16):
                    pltpu.make_async_copy(
                        table_hbm.at[pl.ds(vec[l], 1)],
                        rows_v.at[pl.ds(k * 16 + l, 1)],
                        sem,
                    ).start()
            pltpu.make_async_copy(
                table_hbm.at[pl.ds(0, CH)], rows_v, sem
            ).wait()
            pltpu.sync_copy(rows_v, out_hbm.at[pl.ds(base + j * CH, CH)])

        pl.loop(0, NCH)(chunk)

    return gather_kernel(idx3, table)


TS = 10


def _lstm_body(emb_ref, h0_ref, c0_ref, w_ref, u_ref, b_ref,
               out_ref, hf_ref, cf_ref):
    t = pl.program_id(0)

    @pl.when(t == 0)
    def _():
        hf_ref[...] = h0_ref[...]
        cf_ref[...] = c0_ref[...]

    xw = jnp.dot(emb_ref[...].reshape(TS * B, D), w_ref[...],
                 preferred_element_type=jnp.float32) + b_ref[...]
    h = hf_ref[...]
    c = cf_ref[...]
    for s in range(TS):
        z = (xw[s * B:(s + 1) * B]
             + jnp.dot(h, u_ref[...], preferred_element_type=jnp.float32))
        i = jax.nn.sigmoid(z[:, 0:H])
        f = jax.nn.sigmoid(z[:, H:2 * H])
        g = jnp.tanh(z[:, 2 * H:3 * H])
        o = jax.nn.sigmoid(z[:, 3 * H:4 * H])
        c = f * c + i * g
        h = o * jnp.tanh(c)
        out_ref[s] = h
    hf_ref[...] = h
    cf_ref[...] = c


def _tc_lstm(embed, state_h, state_c, w, u, b2):
    out = pl.pallas_call(
        _lstm_body,
        grid=(T // TS,),
        in_specs=[
            pl.BlockSpec((TS, B, D), lambda t: (t, 0, 0)),
            pl.BlockSpec((B, H), lambda t: (0, 0)),
            pl.BlockSpec((B, H), lambda t: (0, 0)),
            pl.BlockSpec((D, 4 * H), lambda t: (0, 0)),
            pl.BlockSpec((H, 4 * H), lambda t: (0, 0)),
            pl.BlockSpec((1, 4 * H), lambda t: (0, 0)),
        ],
        out_specs=[
            pl.BlockSpec((TS, B, H), lambda t: (t, 0, 0)),
            pl.BlockSpec((B, H), lambda t: (0, 0)),
            pl.BlockSpec((B, H), lambda t: (0, 0)),
        ],
        out_shape=[
            jax.ShapeDtypeStruct((T, B, H), jnp.float32),
            jax.ShapeDtypeStruct((B, H), jnp.float32),
            jax.ShapeDtypeStruct((B, H), jnp.float32),
        ],
    )(embed, state_h, state_c, w, u, b2)
    return out[0], out[1], out[2]


def kernel(sequence, state_h, state_c, embedding, W, U, b):
    idx3 = sequence.astype(jnp.int32).T.reshape(NW, NCH, CH)
    embed = _sc_gather(embedding, idx3).reshape(T, B, D)
    out_t, h_fin, c_fin = _tc_lstm(embed, state_h, state_c, W, U,
                                   b.reshape(1, 4 * H))
    return (out_t.transpose(1, 0, 2), h_fin, c_fin)


# speedup vs baseline: 1.0036x; 1.0036x over previous
import functools

import jax
import jax.numpy as jnp
from jax import lax
from jax.experimental import pallas as pl
from jax.experimental.pallas import tpu as pltpu
from jax.experimental.pallas import tpu_sc as plsc

V = 1000000
D = 64
H = 128
B = 1024
T = 50

NC = 2
NS = 16
NW = NC * NS
N = B * T
BPW = N // NW
CH = 80
NCH = BPW // CH


def _sc_gather(table, idx3):
    mesh = plsc.VectorSubcoreMesh(core_axis_name="c", subcore_axis_name="s")

    @functools.partial(
        pl.kernel,
        mesh=mesh,
        out_type=jax.ShapeDtypeStruct((N, D), jnp.float32),
        scratch_types=[
            pltpu.VMEM((NCH, CH), jnp.int32),
            pltpu.VMEM((CH, D), jnp.float32),
            pltpu.SemaphoreType.DMA,
        ],
    )
    def gather_kernel(idx_hbm, table_hbm, out_hbm, idx_v, rows_v, sem):
        wid = lax.axis_index("s") * NC + lax.axis_index("c")
        base = wid * BPW
        pltpu.sync_copy(idx_hbm.at[wid], idx_v)

        def chunk(j):
            for k in range(CH // 16):
                vec = idx_v[j, pl.ds(k * 16, 16)]
                for l in range(16):
                    pltpu.make_async_copy(
                        table_hbm.at[pl.ds(vec[l], 1)],
                        rows_v.at[pl.ds(k * 16 + l, 1)],
                        sem,
                    ).start()
            pltpu.make_async_copy(
                table_hbm.at[pl.ds(0, CH)], rows_v, sem
            ).wait()
            pltpu.sync_copy(rows_v, out_hbm.at[pl.ds(base + j * CH, CH)])

        pl.loop(0, NCH)(chunk)

    return gather_kernel(idx3, table)


TS = 10


def _lstm_body(emb_ref, h0_ref, c0_ref, w_ref, u_ref, b_ref,
               out_ref, hf_ref, cf_ref):
    t = pl.program_id(0)

    @pl.when(t == 0)
    def _():
        hf_ref[...] = h0_ref[...]
        cf_ref[...] = c0_ref[...]

    xw = jnp.dot(emb_ref[...].reshape(TS * B, D), w_ref[...],
                 preferred_element_type=jnp.float32)
    h = hf_ref[...]
    c = cf_ref[...]
    for s in range(TS):
        z = (xw[s * B:(s + 1) * B]
             + jnp.dot(h, u_ref[...], preferred_element_type=jnp.float32)
             + b_ref[...])
        i = jax.nn.sigmoid(z[:, 0:H])
        f = jax.nn.sigmoid(z[:, H:2 * H])
        g = jnp.tanh(z[:, 2 * H:3 * H])
        o = jax.nn.sigmoid(z[:, 3 * H:4 * H])
        c = f * c + i * g
        h = o * jnp.tanh(c)
        out_ref[s] = h
    hf_ref[...] = h
    cf_ref[...] = c


def _tc_lstm(embed, state_h, state_c, w, u, b2):
    out = pl.pallas_call(
        _lstm_body,
        grid=(T // TS,),
        in_specs=[
            pl.BlockSpec((TS, B, D), lambda t: (t, 0, 0)),
            pl.BlockSpec((B, H), lambda t: (0, 0)),
            pl.BlockSpec((B, H), lambda t: (0, 0)),
            pl.BlockSpec((D, 4 * H), lambda t: (0, 0)),
            pl.BlockSpec((H, 4 * H), lambda t: (0, 0)),
            pl.BlockSpec((1, 4 * H), lambda t: (0, 0)),
        ],
        out_specs=[
            pl.BlockSpec((TS, B, H), lambda t: (t, 0, 0)),
            pl.BlockSpec((B, H), lambda t: (0, 0)),
            pl.BlockSpec((B, H), lambda t: (0, 0)),
        ],
        out_shape=[
            jax.ShapeDtypeStruct((T, B, H), jnp.float32),
            jax.ShapeDtypeStruct((B, H), jnp.float32),
            jax.ShapeDtypeStruct((B, H), jnp.float32),
        ],
    )(embed, state_h, state_c, w, u, b2)
    return out[0], out[1], out[2]


def kernel(sequence, state_h, state_c, embedding, W, U, b):
    idx3 = sequence.astype(jnp.int32).T.reshape(NW, NCH, CH)
    embed = _sc_gather(embedding, idx3).reshape(T, B, D)
    out_t, h_fin, c_fin = _tc_lstm(embed, state_h, state_c, W, U,
                                   b.reshape(1, 4 * H))
    return (out_t.transpose(1, 0, 2), h_fin, c_fin)
